# baseline (speedup 1.0000x reference)
import jax
import jax.numpy as jnp
from jax.experimental import pallas as pl
from jax.experimental.pallas import tpu as pltpu

_CHUNK_ROWS = 256
_NBUF = 6


def _pipelined_scale_copy(w_hbm, o_hbm, vin, vout, in_sems, out_sems, *,
                          scale, n_chunks):
    def in_copy(i):
        slot = i % _NBUF
        return pltpu.make_async_copy(
            w_hbm.at[pl.ds(i * _CHUNK_ROWS, _CHUNK_ROWS), :],
            vin.at[slot],
            in_sems.at[slot],
        )

    def out_copy(i):
        slot = i % _NBUF
        return pltpu.make_async_copy(
            vout.at[slot],
            o_hbm.at[pl.ds(i * _CHUNK_ROWS, _CHUNK_ROWS), :],
            out_sems.at[slot],
        )

    for i in range(min(_NBUF, n_chunks)):
        in_copy(i).start()
    for i in range(n_chunks):
        slot = i % _NBUF
        in_copy(i).wait()
        if i >= _NBUF:
            out_copy(i - _NBUF).wait()
        vout[slot] = vin[slot] * scale
        out_copy(i).start()
        if i + _NBUF < n_chunks:
            in_copy(i + _NBUF).start()
    for i in range(max(0, n_chunks - _NBUF), n_chunks):
        out_copy(i).wait()


def kernel(x, emb_weight):
    seq_len = x.shape[1]
    max_seq, dim = emb_weight.shape
    assert seq_len <= max_seq
    assert seq_len % _CHUNK_ROWS == 0
    scale = dim ** (-0.5)
    n_chunks = seq_len // _CHUNK_ROWS
    import functools
    return pl.pallas_call(
        functools.partial(_pipelined_scale_copy, scale=scale,
                          n_chunks=n_chunks),
        in_specs=[pl.BlockSpec(memory_space=pl.ANY)],
        out_specs=pl.BlockSpec(memory_space=pl.ANY),
        out_shape=jax.ShapeDtypeStruct((seq_len, dim), emb_weight.dtype),
        scratch_shapes=[
            pltpu.VMEM((_NBUF, _CHUNK_ROWS, dim), emb_weight.dtype),
            pltpu.VMEM((_NBUF, _CHUNK_ROWS, dim), emb_weight.dtype),
            pltpu.SemaphoreType.DMA((_NBUF,)),
            pltpu.SemaphoreType.DMA((_NBUF,)),
        ],
        compiler_params=pltpu.CompilerParams(
            vmem_limit_bytes=100 * 1024 * 1024,
        ),
    )(emb_weight)


# speedup vs baseline: 1.0051x; 1.0051x over previous
import jax
import jax.numpy as jnp
from jax.experimental import pallas as pl
from jax.experimental.pallas import tpu as pltpu

_MAX_ROWS = 512
_NBUF = 3
_RAMP = [64, 64, 128, 256]


def _chunk_schedule(seq_len):
    mid_rows = seq_len - 2 * sum(_RAMP)
    assert mid_rows % _MAX_ROWS == 0 and mid_rows > 0
    rows = _RAMP + [_MAX_ROWS] * (mid_rows // _MAX_ROWS) + _RAMP[::-1]
    offs, o = [], 0
    for r in rows:
        offs.append(o)
        o += r
    assert o == seq_len
    return list(zip(offs, rows))


def _pipelined_scale_copy(w_hbm, o_hbm, vin, vout, in_sems, out_sems, *,
                          scale, chunks):
    n_chunks = len(chunks)

    def in_copy(i):
        slot = i % _NBUF
        off, rows = chunks[i]
        return pltpu.make_async_copy(
            w_hbm.at[pl.ds(off, rows), :],
            vin.at[slot, pl.ds(0, rows), :],
            in_sems.at[slot],
        )

    def out_copy(i):
        slot = i % _NBUF
        off, rows = chunks[i]
        return pltpu.make_async_copy(
            vout.at[slot, pl.ds(0, rows), :],
            o_hbm.at[pl.ds(off, rows), :],
            out_sems.at[slot],
        )

    for i in range(min(_NBUF, n_chunks)):
        in_copy(i).start()
    for i in range(n_chunks):
        slot = i % _NBUF
        rows = chunks[i][1]
        in_copy(i).wait()
        if i >= _NBUF:
            out_copy(i - _NBUF).wait()
        vout[slot, 0:rows, :] = vin[slot, 0:rows, :] * scale
        out_copy(i).start()
        if i + _NBUF < n_chunks:
            in_copy(i + _NBUF).start()
    for i in range(max(0, n_chunks - _NBUF), n_chunks):
        out_copy(i).wait()


def kernel(x, emb_weight):
    seq_len = x.shape[1]
    max_seq, dim = emb_weight.shape
    assert seq_len <= max_seq
    scale = dim ** (-0.5)
    chunks = _chunk_schedule(seq_len)
    import functools
    return pl.pallas_call(
        functools.partial(_pipelined_scale_copy, scale=scale, chunks=chunks),
        in_specs=[pl.BlockSpec(memory_space=pl.ANY)],
        out_specs=pl.BlockSpec(memory_space=pl.ANY),
        out_shape=jax.ShapeDtypeStruct((seq_len, dim), emb_weight.dtype),
        scratch_shapes=[
            pltpu.VMEM((_NBUF, _MAX_ROWS, dim), emb_weight.dtype),
            pltpu.VMEM((_NBUF, _MAX_ROWS, dim), emb_weight.dtype),
            pltpu.SemaphoreType.DMA((_NBUF,)),
            pltpu.SemaphoreType.DMA((_NBUF,)),
        ],
        compiler_params=pltpu.CompilerParams(
            vmem_limit_bytes=100 * 1024 * 1024,
        ),
    )(emb_weight)


# speedup vs baseline: 1.0136x; 1.0084x over previous
import jax
import jax.numpy as jnp
from jax.experimental import pallas as pl
from jax.experimental.pallas import tpu as pltpu


def _scale_copy_block(w_ref, o_ref, *, scale):
    o_ref[...] = w_ref[...] * scale


def kernel(x, emb_weight):
    seq_len = x.shape[1]
    max_seq, dim = emb_weight.shape
    assert seq_len <= max_seq
    scale = dim ** (-0.5)
    block_rows = 896
    grid = (pl.cdiv(seq_len, block_rows),)
    import functools
    return pl.pallas_call(
        functools.partial(_scale_copy_block, scale=scale),
        grid=grid,
        in_specs=[pl.BlockSpec((block_rows, dim), lambda i: (i, 0))],
        out_specs=pl.BlockSpec((block_rows, dim), lambda i: (i, 0)),
        out_shape=jax.ShapeDtypeStruct((seq_len, dim), emb_weight.dtype),
        compiler_params=pltpu.CompilerParams(
            vmem_limit_bytes=100 * 1024 * 1024,
        ),
    )(emb_weight)


# speedup vs baseline: 1.0140x; 1.0004x over previous
import functools

import jax
from jax.experimental import pallas as pl
from jax.experimental.pallas import tpu as pltpu

_BLOCK_ROWS = 896


def _scale_copy_block(w_ref, o_ref, *, scale):
    o_ref[...] = w_ref[...] * scale


def kernel(x, emb_weight):
    seq_len = x.shape[1]
    max_seq, dim = emb_weight.shape
    assert seq_len <= max_seq, "input length > max_seq_len"
    scale = dim ** (-0.5)
    block_rows = min(_BLOCK_ROWS, seq_len)
    grid = (pl.cdiv(seq_len, block_rows),)
    return pl.pallas_call(
        functools.partial(_scale_copy_block, scale=scale),
        grid=grid,
        in_specs=[pl.BlockSpec((block_rows, dim), lambda i: (i, 0))],
        out_specs=pl.BlockSpec((block_rows, dim), lambda i: (i, 0)),
        out_shape=jax.ShapeDtypeStruct((seq_len, dim), emb_weight.dtype),
        compiler_params=pltpu.CompilerParams(
            vmem_limit_bytes=100 * 1024 * 1024,
        ),
    )(emb_weight)


# speedup vs baseline: 1.0163x; 1.0022x over previous
import functools

import jax
from jax.experimental import pallas as pl
from jax.experimental.pallas import tpu as pltpu

_BLOCK_ROWS = 960


def _scale_copy_block(w_ref, o_ref, *, scale):
    o_ref[...] = w_ref[...] * scale


def kernel(x, emb_weight):
    seq_len = x.shape[1]
    max_seq, dim = emb_weight.shape
    assert seq_len <= max_seq, "input length > max_seq_len"
    scale = dim ** (-0.5)
    block_rows = min(_BLOCK_ROWS, seq_len)
    grid = (pl.cdiv(seq_len, block_rows),)
    return pl.pallas_call(
        functools.partial(_scale_copy_block, scale=scale),
        grid=grid,
        in_specs=[pl.BlockSpec((block_rows, dim), lambda i: (i, 0))],
        out_specs=pl.BlockSpec((block_rows, dim), lambda i: (i, 0)),
        out_shape=jax.ShapeDtypeStruct((seq_len, dim), emb_weight.dtype),
        compiler_params=pltpu.CompilerParams(
            vmem_limit_bytes=100 * 1024 * 1024,
        ),
    )(emb_weight)
